# baseline (speedup 1.0000x reference)
import functools

import jax
import jax.numpy as jnp
from jax import lax
from jax.experimental import pallas as pl
from jax.experimental.pallas import tpu as pltpu
from jax.experimental.pallas import tpu_sc as plsc

N = 10000
D = 128
E = 320000

NC = 2
NS = 16
NW = NC * NS
CHUNK = 128
NBUF = 2
CHUNKS_PER_TILE = 80
TILE_STRIDE = CHUNKS_PER_TILE
E_PAD = NW * CHUNK * TILE_STRIDE
N_PAD = 10112
ROWS_PER_TILE = N_PAD // NS


def _sc_aggregate(x, src2d, dst2d, zeros):
    mesh = plsc.VectorSubcoreMesh(core_axis_name="c", subcore_axis_name="s")

    @functools.partial(
        pl.kernel,
        mesh=mesh,
        out_type=jax.ShapeDtypeStruct((NC, N_PAD, D), jnp.float32),
        scratch_types=[
            [pltpu.VMEM((CHUNK,), jnp.int32) for _ in range(NBUF)],
            [pltpu.VMEM((CHUNK,), jnp.int32) for _ in range(NBUF)],
            [pltpu.VMEM((CHUNK, D), jnp.float32) for _ in range(NBUF)],
            pltpu.VMEM_SHARED((N_PAD, D), jnp.float32),
            [pltpu.SemaphoreType.DMA for _ in range(NBUF)],
        ],
    )
    def agg_kernel(x_hbm, src_hbm, dst_hbm, zero_hbm, out_hbm,
                   srcv, dstv, rows, acc_sh, gsems):
        c = lax.axis_index("c")
        s = lax.axis_index("s")
        wid = s * NC + c
        r0 = pl.multiple_of(s * ROWS_PER_TILE, 8)
        pltpu.sync_copy(zero_hbm.at[pl.ds(r0, ROWS_PER_TILE)],
                        acc_sh.at[pl.ds(r0, ROWS_PER_TILE)])
        plsc.subcore_barrier()

        base = wid * TILE_STRIDE

        def body(j, carry):
            pltpu.sync_copy(src_hbm.at[base + j], srcv[0])
            pltpu.sync_copy(dst_hbm.at[base + j], dstv[0])
            pltpu.async_copy(x_hbm.at[srcv[0]], rows[0], gsems[0]).wait()
            pltpu.sync_copy(rows[0], acc_sh.at[dstv[0]], add=True)
            return carry

        lax.fori_loop(0, CHUNKS_PER_TILE, body, 0)
        plsc.subcore_barrier()
        pltpu.sync_copy(acc_sh.at[pl.ds(r0, ROWS_PER_TILE)],
                        out_hbm.at[c, pl.ds(r0, ROWS_PER_TILE)])

    return agg_kernel(x, src2d, dst2d, zeros)


def _tc_mlp(z, p0, p1, W1, b1, W2, b2):
    BLK = 1000

    def body(z_ref, p0_ref, p1_ref, W1_ref, b1_ref, W2_ref, b2_ref, o_ref):
        h = z_ref[...] + p0_ref[...] + p1_ref[...]
        h = jnp.maximum(
            jnp.dot(h, W1_ref[...], preferred_element_type=jnp.float32)
            + b1_ref[...], 0.0)
        o_ref[...] = jnp.maximum(
            jnp.dot(h, W2_ref[...], preferred_element_type=jnp.float32)
            + b2_ref[...], 0.0)

    return pl.pallas_call(
        body,
        grid=(N // BLK,),
        in_specs=[
            pl.BlockSpec((BLK, D), lambda i: (i, 0)),
            pl.BlockSpec((BLK, D), lambda i: (i, 0)),
            pl.BlockSpec((BLK, D), lambda i: (i, 0)),
            pl.BlockSpec((D, D), lambda i: (0, 0)),
            pl.BlockSpec((1, D), lambda i: (0, 0)),
            pl.BlockSpec((D, D), lambda i: (0, 0)),
            pl.BlockSpec((1, D), lambda i: (0, 0)),
        ],
        out_specs=pl.BlockSpec((BLK, D), lambda i: (i, 0)),
        out_shape=jax.ShapeDtypeStruct((N, D), jnp.float32),
    )(z, p0, p1, W1, b1.reshape(1, D), W2, b2.reshape(1, D))


def kernel(x, edge_index, W1_0, b1_0, W2_0, b2_0, W1_1, b1_1, W2_1, b2_1):
    pad = E_PAD - E
    pad_dst = N + (jnp.arange(pad, dtype=jnp.int32) % (N_PAD - N))
    src2d = jnp.concatenate(
        [edge_index[0], jnp.zeros((pad,), jnp.int32)]).reshape(-1, CHUNK)
    dst2d = jnp.concatenate([edge_index[1], pad_dst]).reshape(-1, CHUNK)
    zeros = jnp.zeros((N_PAD, D), jnp.float32)

    parts = _sc_aggregate(x, src2d, dst2d, zeros)
    z1 = _tc_mlp(x, parts[0, :N], parts[1, :N], W1_0, b1_0, W2_0, b2_0)
    parts = _sc_aggregate(z1, src2d, dst2d, zeros)
    return _tc_mlp(z1, parts[0, :N], parts[1, :N], W1_1, b1_1, W2_1, b2_1)


# speedup vs baseline: 2.0294x; 2.0294x over previous
import functools

import jax
import jax.numpy as jnp
from jax import lax
from jax.experimental import pallas as pl
from jax.experimental.pallas import tpu as pltpu
from jax.experimental.pallas import tpu_sc as plsc

N = 10000
D = 128
E = 320000

NC = 2
NS = 16
NW = NC * NS
CHUNK = 128
NBUF = 2
CHUNKS_PER_TILE = -(-E // (NW * CHUNK))
TILE_STRIDE = CHUNKS_PER_TILE
E_PAD = NW * CHUNK * TILE_STRIDE
N_PAD = 10112
ROWS_PER_TILE = N_PAD // NS


def _sc_aggregate(x, src2d, dst2d, zeros):
    mesh = plsc.VectorSubcoreMesh(core_axis_name="c", subcore_axis_name="s")

    @functools.partial(
        pl.kernel,
        mesh=mesh,
        out_type=jax.ShapeDtypeStruct((NC, N_PAD, D), jnp.float32),
        scratch_types=[
            [pltpu.VMEM((CHUNK,), jnp.int32) for _ in range(NBUF)],
            [pltpu.VMEM((CHUNK,), jnp.int32) for _ in range(NBUF)],
            [pltpu.VMEM((CHUNK, D), jnp.float32) for _ in range(NBUF)],
            pltpu.VMEM_SHARED((N_PAD, D), jnp.float32),
            [pltpu.SemaphoreType.DMA for _ in range(NBUF)],
        ],
    )
    def agg_kernel(x_hbm, src_hbm, dst_hbm, zero_hbm, out_hbm,
                   srcv, dstv, rows, acc_sh, gsems):
        c = lax.axis_index("c")
        s = lax.axis_index("s")
        wid = s * NC + c
        r0 = pl.multiple_of(s * ROWS_PER_TILE, 8)
        pltpu.sync_copy(zero_hbm.at[pl.ds(r0, ROWS_PER_TILE)],
                        acc_sh.at[pl.ds(r0, ROWS_PER_TILE)])
        plsc.subcore_barrier()

        base = wid * TILE_STRIDE

        def do_chunk(j, b):
            pltpu.sync_copy(src_hbm.at[base + j], srcv[b])
            d = pltpu.async_copy(x_hbm.at[srcv[b]], rows[b], gsems[b])
            pltpu.sync_copy(dst_hbm.at[base + j], dstv[b])
            return d

        def body(g, carry):
            d0 = do_chunk(2 * g, 0)
            d1 = do_chunk(2 * g + 1, 1)
            d0.wait()
            pltpu.sync_copy(rows[0], acc_sh.at[dstv[0]], add=True)
            d1.wait()
            pltpu.sync_copy(rows[1], acc_sh.at[dstv[1]], add=True)
            return carry

        lax.fori_loop(0, CHUNKS_PER_TILE // 2, body, 0)
        dl = do_chunk(CHUNKS_PER_TILE - 1, 0)
        dl.wait()
        pltpu.sync_copy(rows[0], acc_sh.at[dstv[0]], add=True)
        plsc.subcore_barrier()
        pltpu.sync_copy(acc_sh.at[pl.ds(r0, ROWS_PER_TILE)],
                        out_hbm.at[c, pl.ds(r0, ROWS_PER_TILE)])

    return agg_kernel(x, src2d, dst2d, zeros)


def _tc_mlp(z, p0, p1, W1, b1, W2, b2):
    BLK = 1000

    def body(z_ref, p0_ref, p1_ref, W1_ref, b1_ref, W2_ref, b2_ref, o_ref):
        h = z_ref[...] + p0_ref[...] + p1_ref[...]
        h = jnp.maximum(
            jnp.dot(h, W1_ref[...], preferred_element_type=jnp.float32)
            + b1_ref[...], 0.0)
        o_ref[...] = jnp.maximum(
            jnp.dot(h, W2_ref[...], preferred_element_type=jnp.float32)
            + b2_ref[...], 0.0)

    return pl.pallas_call(
        body,
        grid=(N // BLK,),
        in_specs=[
            pl.BlockSpec((BLK, D), lambda i: (i, 0)),
            pl.BlockSpec((BLK, D), lambda i: (i, 0)),
            pl.BlockSpec((BLK, D), lambda i: (i, 0)),
            pl.BlockSpec((D, D), lambda i: (0, 0)),
            pl.BlockSpec((1, D), lambda i: (0, 0)),
            pl.BlockSpec((D, D), lambda i: (0, 0)),
            pl.BlockSpec((1, D), lambda i: (0, 0)),
        ],
        out_specs=pl.BlockSpec((BLK, D), lambda i: (i, 0)),
        out_shape=jax.ShapeDtypeStruct((N, D), jnp.float32),
    )(z, p0, p1, W1, b1.reshape(1, D), W2, b2.reshape(1, D))


def kernel(x, edge_index, W1_0, b1_0, W2_0, b2_0, W1_1, b1_1, W2_1, b2_1):
    pad = E_PAD - E
    pad_dst = N + (jnp.arange(pad, dtype=jnp.int32) % (N_PAD - N))
    src2d = jnp.concatenate(
        [edge_index[0], jnp.zeros((pad,), jnp.int32)]).reshape(-1, CHUNK)
    dst2d = jnp.concatenate([edge_index[1], pad_dst]).reshape(-1, CHUNK)
    zeros = jnp.zeros((N_PAD, D), jnp.float32)

    parts = _sc_aggregate(x, src2d, dst2d, zeros)
    z1 = _tc_mlp(x, parts[0, :N], parts[1, :N], W1_0, b1_0, W2_0, b2_0)
    parts = _sc_aggregate(z1, src2d, dst2d, zeros)
    return _tc_mlp(z1, parts[0, :N], parts[1, :N], W1_1, b1_1, W2_1, b2_1)


# speedup vs baseline: 2.0758x; 1.0229x over previous
import functools

import jax
import jax.numpy as jnp
from jax import lax
from jax.experimental import pallas as pl
from jax.experimental.pallas import tpu as pltpu
from jax.experimental.pallas import tpu_sc as plsc

N = 10000
D = 128
E = 320000

NC = 2
NS = 16
NW = NC * NS
CHUNK = 128
NBUF = 3
CHUNKS_PER_TILE = -(-E // (NW * CHUNK))
TILE_STRIDE = CHUNKS_PER_TILE
E_PAD = NW * CHUNK * TILE_STRIDE
N_PAD = 10112
ROWS_PER_TILE = N_PAD // NS


def _sc_aggregate(x, src2d, dst2d, zeros):
    mesh = plsc.VectorSubcoreMesh(core_axis_name="c", subcore_axis_name="s")

    @functools.partial(
        pl.kernel,
        mesh=mesh,
        out_type=jax.ShapeDtypeStruct((NC, N_PAD, D), jnp.float32),
        scratch_types=[
            [pltpu.VMEM((CHUNK,), jnp.int32) for _ in range(NBUF)],
            [pltpu.VMEM((CHUNK,), jnp.int32) for _ in range(NBUF)],
            [pltpu.VMEM((CHUNK, D), jnp.float32) for _ in range(NBUF)],
            pltpu.VMEM_SHARED((N_PAD, D), jnp.float32),
            [pltpu.SemaphoreType.DMA for _ in range(NBUF)],
        ],
    )
    def agg_kernel(x_hbm, src_hbm, dst_hbm, zero_hbm, out_hbm,
                   srcv, dstv, rows, acc_sh, gsems):
        c = lax.axis_index("c")
        s = lax.axis_index("s")
        wid = s * NC + c
        r0 = pl.multiple_of(s * ROWS_PER_TILE, 8)
        pltpu.sync_copy(zero_hbm.at[pl.ds(r0, ROWS_PER_TILE)],
                        acc_sh.at[pl.ds(r0, ROWS_PER_TILE)])
        plsc.subcore_barrier()

        base = wid * TILE_STRIDE

        def do_chunk(j, b):
            pltpu.sync_copy(src_hbm.at[base + j], srcv[b])
            d = pltpu.async_copy(x_hbm.at[srcv[b]], rows[b], gsems[b])
            pltpu.sync_copy(dst_hbm.at[base + j], dstv[b])
            return d

        def body(g, carry):
            descs = [do_chunk(NBUF * g + b, b) for b in range(NBUF)]
            for b in range(NBUF):
                descs[b].wait()
                pltpu.sync_copy(rows[b], acc_sh.at[dstv[b]], add=True)
            return carry

        nfull = CHUNKS_PER_TILE // NBUF
        lax.fori_loop(0, nfull, body, 0)
        for b in range(CHUNKS_PER_TILE - NBUF * nfull):
            dl = do_chunk(NBUF * nfull + b, b)
            dl.wait()
            pltpu.sync_copy(rows[b], acc_sh.at[dstv[b]], add=True)
        plsc.subcore_barrier()
        pltpu.sync_copy(acc_sh.at[pl.ds(r0, ROWS_PER_TILE)],
                        out_hbm.at[c, pl.ds(r0, ROWS_PER_TILE)])

    return agg_kernel(x, src2d, dst2d, zeros)


def _tc_mlp(z, p0, p1, W1, b1, W2, b2):
    BLK = 1000

    def body(z_ref, p0_ref, p1_ref, W1_ref, b1_ref, W2_ref, b2_ref, o_ref):
        h = z_ref[...] + p0_ref[...] + p1_ref[...]
        h = jnp.maximum(
            jnp.dot(h, W1_ref[...], preferred_element_type=jnp.float32)
            + b1_ref[...], 0.0)
        o_ref[...] = jnp.maximum(
            jnp.dot(h, W2_ref[...], preferred_element_type=jnp.float32)
            + b2_ref[...], 0.0)

    return pl.pallas_call(
        body,
        grid=(N // BLK,),
        in_specs=[
            pl.BlockSpec((BLK, D), lambda i: (i, 0)),
            pl.BlockSpec((BLK, D), lambda i: (i, 0)),
            pl.BlockSpec((BLK, D), lambda i: (i, 0)),
            pl.BlockSpec((D, D), lambda i: (0, 0)),
            pl.BlockSpec((1, D), lambda i: (0, 0)),
            pl.BlockSpec((D, D), lambda i: (0, 0)),
            pl.BlockSpec((1, D), lambda i: (0, 0)),
        ],
        out_specs=pl.BlockSpec((BLK, D), lambda i: (i, 0)),
        out_shape=jax.ShapeDtypeStruct((N, D), jnp.float32),
    )(z, p0, p1, W1, b1.reshape(1, D), W2, b2.reshape(1, D))


def kernel(x, edge_index, W1_0, b1_0, W2_0, b2_0, W1_1, b1_1, W2_1, b2_1):
    pad = E_PAD - E
    pad_dst = N + (jnp.arange(pad, dtype=jnp.int32) % (N_PAD - N))
    src2d = jnp.concatenate(
        [edge_index[0], jnp.zeros((pad,), jnp.int32)]).reshape(-1, CHUNK)
    dst2d = jnp.concatenate([edge_index[1], pad_dst]).reshape(-1, CHUNK)
    zeros = jnp.zeros((N_PAD, D), jnp.float32)

    parts = _sc_aggregate(x, src2d, dst2d, zeros)
    z1 = _tc_mlp(x, parts[0, :N], parts[1, :N], W1_0, b1_0, W2_0, b2_0)
    parts = _sc_aggregate(z1, src2d, dst2d, zeros)
    return _tc_mlp(z1, parts[0, :N], parts[1, :N], W1_1, b1_1, W2_1, b2_1)


# speedup vs baseline: 2.0769x; 1.0005x over previous
import functools

import jax
import jax.numpy as jnp
from jax import lax
from jax.experimental import pallas as pl
from jax.experimental.pallas import tpu as pltpu
from jax.experimental.pallas import tpu_sc as plsc

N = 10000
D = 128
E = 320000

NC = 2
NS = 16
NW = NC * NS
CHUNK = 128
NBUF = 3
CHUNKS_PER_TILE = -(-E // (NW * CHUNK))
TILE_STRIDE = CHUNKS_PER_TILE
E_PAD = NW * CHUNK * TILE_STRIDE
N_PAD = 10112
ROWS_PER_TILE = N_PAD // NS


def _sc_aggregate(x, src2d, dst2d, zeros):
    mesh = plsc.VectorSubcoreMesh(core_axis_name="c", subcore_axis_name="s")

    @functools.partial(
        pl.kernel,
        mesh=mesh,
        out_type=jax.ShapeDtypeStruct((NC, N_PAD, D), jnp.float32),
        scratch_types=[
            [pltpu.VMEM((CHUNK,), jnp.int32) for _ in range(NBUF)],
            [pltpu.VMEM((CHUNK,), jnp.int32) for _ in range(NBUF)],
            [pltpu.VMEM((CHUNK, D), jnp.float32) for _ in range(NBUF)],
            pltpu.VMEM_SHARED((N_PAD, D), jnp.float32),
            [pltpu.SemaphoreType.DMA for _ in range(NBUF)],
        ],
    )
    def agg_kernel(x_hbm, src_hbm, dst_hbm, zero_hbm, out_hbm,
                   srcv, dstv, rows, acc_sh, gsems):
        c = lax.axis_index("c")
        s = lax.axis_index("s")
        wid = s * NC + c
        r0 = pl.multiple_of(s * ROWS_PER_TILE, 8)
        pltpu.sync_copy(zero_hbm.at[pl.ds(r0, ROWS_PER_TILE)],
                        acc_sh.at[pl.ds(r0, ROWS_PER_TILE)])
        plsc.subcore_barrier()

        base = wid * TILE_STRIDE

        def do_chunk(j, b):
            pltpu.sync_copy(src_hbm.at[base + j], srcv[b])
            d = pltpu.async_copy(x_hbm.at[srcv[b]], rows[b], gsems[b])
            pltpu.sync_copy(dst_hbm.at[base + j], dstv[b])
            return d

        def body(g, carry):
            descs = [do_chunk(NBUF * g + b, b) for b in range(NBUF)]
            for b in range(NBUF):
                descs[b].wait()
                pltpu.sync_copy(rows[b], acc_sh.at[dstv[b]], add=True)
            return carry

        nfull = CHUNKS_PER_TILE // NBUF
        lax.fori_loop(0, nfull, body, 0)
        for b in range(CHUNKS_PER_TILE - NBUF * nfull):
            dl = do_chunk(NBUF * nfull + b, b)
            dl.wait()
            pltpu.sync_copy(rows[b], acc_sh.at[dstv[b]], add=True)
        plsc.subcore_barrier()
        pltpu.sync_copy(acc_sh.at[pl.ds(r0, ROWS_PER_TILE)],
                        out_hbm.at[c, pl.ds(r0, ROWS_PER_TILE)])

    return agg_kernel(x, src2d, dst2d, zeros)


def _tc_mlp(z, parts, W1, b1, W2, b2):
    BLK = 1000

    def body(z_ref, p0_ref, p1_ref, W1_ref, b1_ref, W2_ref, b2_ref, o_ref):
        h = z_ref[...] + p0_ref[0] + p1_ref[0]
        h = jnp.maximum(
            jnp.dot(h, W1_ref[...], preferred_element_type=jnp.float32)
            + b1_ref[...], 0.0)
        o_ref[...] = jnp.maximum(
            jnp.dot(h, W2_ref[...], preferred_element_type=jnp.float32)
            + b2_ref[...], 0.0)

    return pl.pallas_call(
        body,
        grid=(N // BLK,),
        in_specs=[
            pl.BlockSpec((BLK, D), lambda i: (i, 0)),
            pl.BlockSpec((1, BLK, D), lambda i: (0, i, 0)),
            pl.BlockSpec((1, BLK, D), lambda i: (1, i, 0)),
            pl.BlockSpec((D, D), lambda i: (0, 0)),
            pl.BlockSpec((1, D), lambda i: (0, 0)),
            pl.BlockSpec((D, D), lambda i: (0, 0)),
            pl.BlockSpec((1, D), lambda i: (0, 0)),
        ],
        out_specs=pl.BlockSpec((BLK, D), lambda i: (i, 0)),
        out_shape=jax.ShapeDtypeStruct((N, D), jnp.float32),
    )(z, parts, parts, W1, b1.reshape(1, D), W2, b2.reshape(1, D))


def kernel(x, edge_index, W1_0, b1_0, W2_0, b2_0, W1_1, b1_1, W2_1, b2_1):
    pad = E_PAD - E
    pad_dst = N + (jnp.arange(pad, dtype=jnp.int32) % (N_PAD - N))
    src2d = jnp.concatenate(
        [edge_index[0], jnp.zeros((pad,), jnp.int32)]).reshape(-1, CHUNK)
    dst2d = jnp.concatenate([edge_index[1], pad_dst]).reshape(-1, CHUNK)
    zeros = jnp.zeros((N_PAD, D), jnp.float32)

    parts = _sc_aggregate(x, src2d, dst2d, zeros)
    z1 = _tc_mlp(x, parts, W1_0, b1_0, W2_0, b2_0)
    parts = _sc_aggregate(z1, src2d, dst2d, zeros)
    return _tc_mlp(z1, parts, W1_1, b1_1, W2_1, b2_1)
